# Initial kernel scaffold; baseline (speedup 1.0000x reference)
import jax
import jax.numpy as jnp
from jax.experimental import pallas as pl


def kernel(x, pos, edge_index_inter, W_n0, b_n0, g0, be0, rm0, rv0, W_rbf, b_rbf, W_r1, b_r1, W_r2, b_r2, W_c1, b_c1, W_c2, b_c2, W_n, b_n, g1, be1, rm1, rv1):
    raise NotImplementedError("write your pallas kernel here")


# speedup vs baseline: 4.4355x; 4.4355x over previous
import functools

import jax
import jax.numpy as jnp
from jax import lax
from jax.experimental import pallas as pl
from jax.experimental.pallas import tpu as pltpu
from jax.experimental.pallas import tpu_sc as plsc

N = 10000
E = 320000
D = 128

NC = 2
NS = 16
NW = NC * NS

GCHUNK = 128
GSPAN = 10112
GCH_FULL = GSPAN // GCHUNK
GCH_LAST = (E - (NW - 1) * GSPAN) // GCHUNK

SC_EDGES = E // NC
T_EDGES = SC_EDGES // NS
SCH_FULL = T_EDGES // GCHUNK
SREM = T_EDGES - SCH_FULL * GCHUNK
RPT = 624
RPT_LAST = N - 15 * RPT

_sds = jax.ShapeDtypeStruct


def _leaky(v):
    return jnp.where(v >= 0, v, 0.01 * v)


def _sigmoid(v):
    return 1.0 / (1.0 + jnp.exp(-v))


def _node0_body(x_ref, w_ref, p_ref, h_ref):
    b = p_ref[0, :]
    g = p_ref[1, :]
    be = p_ref[2, :]
    rm = p_ref[3, :]
    rv = p_ref[4, :]
    scale = g / jnp.sqrt(rv + 1e-5)
    y = jnp.dot(x_ref[...], w_ref[...], preferred_element_type=jnp.float32) + b
    y = _leaky(y)
    h_ref[...] = (y - rm) * scale + be


def _node0(x, w, params):
    blk = 1000
    return pl.pallas_call(
        _node0_body,
        grid=(N // blk,),
        in_specs=[
            pl.BlockSpec((blk, D), lambda i: (i, 0)),
            pl.BlockSpec((D, D), lambda i: (0, 0)),
            pl.BlockSpec((5, D), lambda i: (0, 0)),
        ],
        out_specs=pl.BlockSpec((blk, D), lambda i: (i, 0)),
        out_shape=_sds((N, D), jnp.float32),
    )(x, w, params)


def _sc_sqrt(x):
    bits = plsc.bitcast(x, jnp.int32)
    y = plsc.bitcast((bits >> 1) + 0x1FBD1DF5, jnp.float32)
    y = 0.5 * (y + x / y)
    y = 0.5 * (y + x / y)
    y = 0.5 * (y + x / y)
    return y


def _gather_body(h_hbm, px_hbm, py_hbm, pz_hbm, row_hbm, col_hbm,
                 hr_hbm, hc_hbm, rbf_hbm,
                 idxr_v, idxc_v, hr_v, hc_v, rbf_v, px_v, py_v, pz_v, sem):
    cid = lax.axis_index("c")
    sid = lax.axis_index("s")
    wid = sid * NC + cid
    base_w = wid * GSPAN
    nch = jnp.where(wid == NW - 1, GCH_LAST, GCH_FULL)

    pltpu.sync_copy(px_hbm, px_v)
    pltpu.sync_copy(py_hbm, py_v)
    pltpu.sync_copy(pz_hbm, pz_v)

    def chunk(i, carry):
        base = base_w + i * GCHUNK
        pltpu.sync_copy(row_hbm.at[pl.ds(base, GCHUNK)], idxr_v)
        pltpu.sync_copy(col_hbm.at[pl.ds(base, GCHUNK)], idxc_v)
        d1 = pltpu.async_copy(h_hbm.at[idxr_v], hr_v, sem)
        d2 = pltpu.async_copy(h_hbm.at[idxc_v], hc_v, sem)
        for g in range(GCHUNK // 16):
            ir = idxr_v[pl.ds(g * 16, 16)]
            ic = idxc_v[pl.ds(g * 16, 16)]
            dx = plsc.load_gather(px_v, [ir]) - plsc.load_gather(px_v, [ic])
            dy = plsc.load_gather(py_v, [ir]) - plsc.load_gather(py_v, [ic])
            dz = plsc.load_gather(pz_v, [ir]) - plsc.load_gather(pz_v, [ic])
            dist = _sc_sqrt(dx * dx + dy * dy + dz * dz + 1e-12)
            rid = g * 16 + lax.iota(jnp.int32, 16)
            for j in range(9):
                z = (dist - (2.5 * j)) * 0.45
                plsc.store_scatter(rbf_v, [rid, jnp.full((16,), j, jnp.int32)],
                                   jnp.exp(-(z * z)))
        d1.wait()
        d2.wait()
        pltpu.sync_copy(hr_v, hr_hbm.at[pl.ds(base, GCHUNK)])
        pltpu.sync_copy(hc_v, hc_hbm.at[pl.ds(base, GCHUNK)])
        pltpu.sync_copy(rbf_v, rbf_hbm.at[pl.ds(base, GCHUNK)])
        return carry

    lax.fori_loop(0, nch, chunk, 0)


def _gather(h, px, py, pz, row, col):
    mesh = plsc.VectorSubcoreMesh(core_axis_name="c", subcore_axis_name="s",
                                  num_cores=NC, num_subcores=NS)
    k = pl.kernel(
        _gather_body,
        out_type=(
            _sds((E, D), jnp.float32),
            _sds((E, D), jnp.float32),
            _sds((E, 16), jnp.float32),
        ),
        mesh=mesh,
        scratch_types=[
            pltpu.VMEM((GCHUNK,), jnp.int32),
            pltpu.VMEM((GCHUNK,), jnp.int32),
            pltpu.VMEM((GCHUNK, D), jnp.float32),
            pltpu.VMEM((GCHUNK, D), jnp.float32),
            pltpu.VMEM((GCHUNK, 16), jnp.float32),
            pltpu.VMEM((N,), jnp.float32),
            pltpu.VMEM((N,), jnp.float32),
            pltpu.VMEM((N,), jnp.float32),
            pltpu.SemaphoreType.DMA,
        ],
        compiler_params=pltpu.CompilerParams(needs_layout_passes=False),
    )
    return k(h, px, py, pz, row, col)


EBLK = 4000


def _edge_body(hr_ref, hc_ref, rbf_ref,
               wrbf_ref, wr1_ref, wr2_ref, wc1_ref, wc2_ref, bias_ref,
               ef_ref):
    hr = hr_ref[...]
    hc = hc_ref[...]
    rbf = rbf_ref[...][:, 0:9]

    b_rbf = bias_ref[0, :]
    b_r1 = bias_ref[1, :]
    b_r2 = bias_ref[2, :]
    b_c1 = bias_ref[3, :]
    b_c2 = bias_ref[4, :]

    rad = jnp.dot(rbf, wrbf_ref[...], preferred_element_type=jnp.float32) + b_rbf
    rad = rad * _sigmoid(rad)

    def mlp(w1_ref, b1, w2_ref, b2):
        pre = (jnp.dot(hr, w1_ref[0:D, :], preferred_element_type=jnp.float32)
               + jnp.dot(hc, w1_ref[D:2 * D, :], preferred_element_type=jnp.float32)
               + jnp.dot(rad, w1_ref[2 * D:3 * D, :], preferred_element_type=jnp.float32)
               + b1)
        t = pre * _sigmoid(pre)
        return jnp.dot(t, w2_ref[...], preferred_element_type=jnp.float32) + b2

    row_out = mlp(wr1_ref, b_r1, wr2_ref, b_r2)
    col_out = mlp(wc1_ref, b_c1, wc2_ref, b_c2)
    ef_ref[...] = hr * _sigmoid(row_out) + hc * _sigmoid(col_out)


def _edge_mlp(hr, hc, rbf, wrbf, wr1, wr2, wc1, wc2, biases):
    return pl.pallas_call(
        _edge_body,
        grid=(E // EBLK,),
        in_specs=[
            pl.BlockSpec((EBLK, D), lambda i: (i, 0)),
            pl.BlockSpec((EBLK, D), lambda i: (i, 0)),
            pl.BlockSpec((EBLK, 16), lambda i: (i, 0)),
            pl.BlockSpec((9, D), lambda i: (0, 0)),
            pl.BlockSpec((3 * D, D), lambda i: (0, 0)),
            pl.BlockSpec((D, D), lambda i: (0, 0)),
            pl.BlockSpec((3 * D, D), lambda i: (0, 0)),
            pl.BlockSpec((D, D), lambda i: (0, 0)),
            pl.BlockSpec((5, D), lambda i: (0, 0)),
        ],
        out_specs=pl.BlockSpec((EBLK, D), lambda i: (i, 0)),
        out_shape=_sds((E, D), jnp.float32),
    )(hr, hc, rbf, wrbf, wr1, wr2, wc1, wc2, biases)


def _scatter_body(ef_hbm, row_hbm, zero_hbm, out_hbm,
                  idx_v, feat_v, idx2_v, feat2_v, agg_sh, sem):
    cid = lax.axis_index("c")
    sid = lax.axis_index("s")
    rbase = sid * RPT

    @pl.when(sid < NS - 1)
    def _():
        pltpu.sync_copy(zero_hbm.at[pl.ds(rbase, RPT)],
                        agg_sh.at[pl.ds(rbase, RPT)])

    @pl.when(sid == NS - 1)
    def _():
        pltpu.sync_copy(zero_hbm.at[pl.ds(rbase, RPT_LAST)],
                        agg_sh.at[pl.ds(rbase, RPT_LAST)])

    plsc.subcore_barrier()

    base_t = cid * SC_EDGES + sid * T_EDGES

    def chunk(i, carry):
        base = base_t + i * GCHUNK
        pltpu.sync_copy(row_hbm.at[pl.ds(base, GCHUNK)], idx_v)
        pltpu.sync_copy(ef_hbm.at[pl.ds(base, GCHUNK)], feat_v)
        pltpu.sync_copy(feat_v, agg_sh.at[idx_v], add=True)
        return carry

    lax.fori_loop(0, SCH_FULL, chunk, 0)

    base = base_t + SCH_FULL * GCHUNK
    pltpu.sync_copy(row_hbm.at[pl.ds(base, SREM)], idx2_v)
    pltpu.sync_copy(ef_hbm.at[pl.ds(base, SREM)], feat2_v)
    pltpu.sync_copy(feat2_v, agg_sh.at[idx2_v], add=True)

    plsc.subcore_barrier()

    @pl.when(sid < NS - 1)
    def _():
        pltpu.sync_copy(agg_sh.at[pl.ds(rbase, RPT)],
                        out_hbm.at[cid, pl.ds(rbase, RPT)])

    @pl.when(sid == NS - 1)
    def _():
        pltpu.sync_copy(agg_sh.at[pl.ds(rbase, RPT_LAST)],
                        out_hbm.at[cid, pl.ds(rbase, RPT_LAST)])


def _scatter(edge_feat, row, zeros):
    mesh = plsc.VectorSubcoreMesh(core_axis_name="c", subcore_axis_name="s",
                                  num_cores=NC, num_subcores=NS)
    k = pl.kernel(
        _scatter_body,
        out_type=_sds((NC, N, D), jnp.float32),
        mesh=mesh,
        scratch_types=[
            pltpu.VMEM((GCHUNK,), jnp.int32),
            pltpu.VMEM((GCHUNK, D), jnp.float32),
            pltpu.VMEM((SREM,), jnp.int32),
            pltpu.VMEM((SREM, D), jnp.float32),
            pltpu.VMEM_SHARED((N, D), jnp.float32),
            pltpu.SemaphoreType.DMA,
        ],
        compiler_params=pltpu.CompilerParams(needs_layout_passes=False),
    )
    return k(edge_feat, row, zeros)


def _node1_body(h_ref, ap_ref, w_ref, p_ref, o_ref):
    b = p_ref[0, :]
    g = p_ref[1, :]
    be = p_ref[2, :]
    rm = p_ref[3, :]
    rv = p_ref[4, :]
    scale = g / jnp.sqrt(rv + 1e-5)
    h = h_ref[...]
    agg = ap_ref[0] + ap_ref[1]
    y = (jnp.dot(h, w_ref[0:D, :], preferred_element_type=jnp.float32)
         + jnp.dot(agg, w_ref[D:2 * D, :], preferred_element_type=jnp.float32)
         + b)
    y = _leaky(y)
    o_ref[...] = h + (y - rm) * scale + be


def _node1(h, agg_partials, w, params):
    blk = 1000
    return pl.pallas_call(
        _node1_body,
        grid=(N // blk,),
        in_specs=[
            pl.BlockSpec((blk, D), lambda i: (i, 0)),
            pl.BlockSpec((NC, blk, D), lambda i: (0, i, 0)),
            pl.BlockSpec((2 * D, D), lambda i: (0, 0)),
            pl.BlockSpec((5, D), lambda i: (0, 0)),
        ],
        out_specs=pl.BlockSpec((blk, D), lambda i: (i, 0)),
        out_shape=_sds((N, D), jnp.float32),
    )(h, agg_partials, w, params)


def kernel(x, pos, edge_index_inter, W_n0, b_n0, g0, be0, rm0, rv0,
           W_rbf, b_rbf, W_r1, b_r1, W_r2, b_r2, W_c1, b_c1, W_c2, b_c2,
           W_n, b_n, g1, be1, rm1, rv1):
    row = edge_index_inter[0]
    col = edge_index_inter[1]
    px, py, pz = pos[:, 0], pos[:, 1], pos[:, 2]
    params0 = jnp.stack([b_n0, g0, be0, rm0, rv0])
    params1 = jnp.stack([b_n, g1, be1, rm1, rv1])
    biases = jnp.stack([b_rbf, b_r1, b_r2, b_c1, b_c2])
    zeros = jnp.zeros((N, D), jnp.float32)

    h = _node0(x, W_n0, params0)
    hr, hc, rbf = _gather(h, px, py, pz, row, col)
    edge_feat = _edge_mlp(hr, hc, rbf, W_rbf, W_r1, W_r2, W_c1, W_c2, biases)
    agg_partials = _scatter(edge_feat, row, zeros)
    return _node1(h, agg_partials, W_n, params1)


# speedup vs baseline: 6.1390x; 1.3841x over previous
import functools

import jax
import jax.numpy as jnp
from jax import lax
from jax.experimental import pallas as pl
from jax.experimental.pallas import tpu as pltpu
from jax.experimental.pallas import tpu_sc as plsc

N = 10000
E = 320000
D = 128

NC = 2
NS = 16
NW = NC * NS

GCHUNK = 128
GSPAN = 10112
GCH_FULL = GSPAN // GCHUNK
GCH_LAST = (E - (NW - 1) * GSPAN) // GCHUNK

SC_EDGES = E // NC
T_EDGES = SC_EDGES // NS
SCH_FULL = T_EDGES // GCHUNK
SREM = T_EDGES - SCH_FULL * GCHUNK
RPT = 624
RPT_LAST = N - 15 * RPT

_sds = jax.ShapeDtypeStruct


def _leaky(v):
    return jnp.where(v >= 0, v, 0.01 * v)


def _sigmoid(v):
    return 1.0 / (1.0 + jnp.exp(-v))


def _node0_body(x_ref, w_ref, p_ref, h_ref):
    b = p_ref[0, :]
    g = p_ref[1, :]
    be = p_ref[2, :]
    rm = p_ref[3, :]
    rv = p_ref[4, :]
    scale = g / jnp.sqrt(rv + 1e-5)
    y = jnp.dot(x_ref[...], w_ref[...], preferred_element_type=jnp.float32) + b
    y = _leaky(y)
    h_ref[...] = (y - rm) * scale + be


def _node0(x, w, params):
    blk = 1000
    return pl.pallas_call(
        _node0_body,
        grid=(N // blk,),
        in_specs=[
            pl.BlockSpec((blk, D), lambda i: (i, 0)),
            pl.BlockSpec((D, D), lambda i: (0, 0)),
            pl.BlockSpec((5, D), lambda i: (0, 0)),
        ],
        out_specs=pl.BlockSpec((blk, D), lambda i: (i, 0)),
        out_shape=_sds((N, D), jnp.float32),
    )(x, w, params)


def _sc_sqrt(x):
    bits = plsc.bitcast(x, jnp.int32)
    y = plsc.bitcast((bits >> 1) + 0x1FBD1DF5, jnp.float32)
    y = 0.5 * (y + x / y)
    y = 0.5 * (y + x / y)
    y = 0.5 * (y + x / y)
    return y


def _gather_body(h_hbm, px_hbm, py_hbm, pz_hbm, row_hbm, col_hbm,
                 hr_hbm, hc_hbm, rbf_hbm,
                 idxr_v, idxc_v, hr_v, hc_v, rbf_v, px_v, py_v, pz_v,
                 gsems, ssems):
    cid = lax.axis_index("c")
    sid = lax.axis_index("s")
    wid = sid * NC + cid
    base_w = wid * GSPAN
    nch = jnp.where(wid == NW - 1, GCH_LAST, GCH_FULL)

    pltpu.sync_copy(px_hbm, px_v)
    pltpu.sync_copy(py_hbm, py_v)
    pltpu.sync_copy(pz_hbm, pz_v)

    def load_and_fire(i, b):
        base = base_w + i * GCHUNK
        pltpu.sync_copy(row_hbm.at[pl.ds(base, GCHUNK)], idxr_v.at[b])
        pltpu.sync_copy(col_hbm.at[pl.ds(base, GCHUNK)], idxc_v.at[b])
        pltpu.async_copy(h_hbm.at[idxr_v.at[b]], hr_v.at[b], gsems.at[b])
        pltpu.async_copy(h_hbm.at[idxc_v.at[b]], hc_v.at[b], gsems.at[b])

    def drain_stores(b):
        pltpu.make_async_copy(hr_v.at[b], hr_hbm.at[pl.ds(0, GCHUNK)],
                              ssems.at[b]).wait()
        pltpu.make_async_copy(hc_v.at[b], hc_hbm.at[pl.ds(0, GCHUNK)],
                              ssems.at[b]).wait()
        pltpu.make_async_copy(rbf_v.at[b], rbf_hbm.at[pl.ds(0, GCHUNK)],
                              ssems.at[b]).wait()

    def drain_gathers(b):
        pltpu.make_async_copy(h_hbm.at[idxr_v.at[b]], hr_v.at[b],
                              gsems.at[b]).wait()
        pltpu.make_async_copy(h_hbm.at[idxc_v.at[b]], hc_v.at[b],
                              gsems.at[b]).wait()

    def compute_rbf(b):
        for g in range(GCHUNK // 16):
            ir = idxr_v[b, pl.ds(g * 16, 16)]
            ic = idxc_v[b, pl.ds(g * 16, 16)]
            dx = plsc.load_gather(px_v, [ir]) - plsc.load_gather(px_v, [ic])
            dy = plsc.load_gather(py_v, [ir]) - plsc.load_gather(py_v, [ic])
            dz = plsc.load_gather(pz_v, [ir]) - plsc.load_gather(pz_v, [ic])
            dist = _sc_sqrt(dx * dx + dy * dy + dz * dz + 1e-12)
            rid = g * 16 + lax.iota(jnp.int32, 16)
            for j in range(9):
                z = (dist - (2.5 * j)) * 0.45
                plsc.store_scatter(rbf_v.at[b],
                                   [rid, jnp.full((16,), j, jnp.int32)],
                                   jnp.exp(-(z * z)))

    def fire_stores(i, b):
        base = base_w + i * GCHUNK
        pltpu.async_copy(hr_v.at[b], hr_hbm.at[pl.ds(base, GCHUNK)],
                         ssems.at[b])
        pltpu.async_copy(hc_v.at[b], hc_hbm.at[pl.ds(base, GCHUNK)],
                         ssems.at[b])
        pltpu.async_copy(rbf_v.at[b], rbf_hbm.at[pl.ds(base, GCHUNK)],
                         ssems.at[b])

    load_and_fire(0, 0)

    def half(i, b):
        @pl.when(i < nch)
        def _():
            @pl.when(i + 1 < nch)
            def _():
                @pl.when(i >= 1)
                def _():
                    drain_stores(1 - b)

                load_and_fire(i + 1, 1 - b)

            compute_rbf(b)
            drain_gathers(b)
            fire_stores(i, b)

    def pair(k, carry):
        half(2 * k, 0)
        half(2 * k + 1, 1)
        return carry

    lax.fori_loop(0, (GCH_FULL + 1) // 2, pair, 0)

    drain_stores(1)
    drain_stores(0)


def _gather(h, px, py, pz, row, col):
    mesh = plsc.VectorSubcoreMesh(core_axis_name="c", subcore_axis_name="s",
                                  num_cores=NC, num_subcores=NS)
    k = pl.kernel(
        _gather_body,
        out_type=(
            _sds((E, D), jnp.float32),
            _sds((E, D), jnp.float32),
            _sds((E, 16), jnp.float32),
        ),
        mesh=mesh,
        scratch_types=[
            pltpu.VMEM((2, GCHUNK), jnp.int32),
            pltpu.VMEM((2, GCHUNK), jnp.int32),
            pltpu.VMEM((2, GCHUNK, D), jnp.float32),
            pltpu.VMEM((2, GCHUNK, D), jnp.float32),
            pltpu.VMEM((2, GCHUNK, 16), jnp.float32),
            pltpu.VMEM((N,), jnp.float32),
            pltpu.VMEM((N,), jnp.float32),
            pltpu.VMEM((N,), jnp.float32),
            pltpu.SemaphoreType.DMA((2,)),
            pltpu.SemaphoreType.DMA((2,)),
        ],
        compiler_params=pltpu.CompilerParams(needs_layout_passes=False),
    )
    return k(h, px, py, pz, row, col)


EBLK = 4000


def _edge_body(hr_ref, hc_ref, rbf_ref,
               wrbf_ref, w1_ref, w2_ref, brbf_ref, bcat_ref,
               ef_ref):
    hr = hr_ref[...]
    hc = hc_ref[...]
    rbf = rbf_ref[...][:, 0:9]

    rad = (jnp.dot(rbf, wrbf_ref[...], preferred_element_type=jnp.float32)
           + brbf_ref[0, :])
    rad = rad * _sigmoid(rad)

    lhs = jnp.concatenate([hr, hc, rad], axis=1).astype(jnp.bfloat16)
    pre = (jnp.dot(lhs, w1_ref[...], preferred_element_type=jnp.float32)
           + bcat_ref[0, :])
    t = (pre * _sigmoid(pre)).astype(jnp.bfloat16)
    out2 = (jnp.dot(t, w2_ref[...], preferred_element_type=jnp.float32)
            + bcat_ref[1, :])
    row_out = out2[:, 0:D]
    col_out = out2[:, D:2 * D]
    ef_ref[...] = hr * _sigmoid(row_out) + hc * _sigmoid(col_out)


def _edge_mlp(hr, hc, rbf, wrbf, w1cat, w2diag, brbf, bcat):
    return pl.pallas_call(
        _edge_body,
        grid=(E // EBLK,),
        in_specs=[
            pl.BlockSpec((EBLK, D), lambda i: (i, 0)),
            pl.BlockSpec((EBLK, D), lambda i: (i, 0)),
            pl.BlockSpec((EBLK, 16), lambda i: (i, 0)),
            pl.BlockSpec((9, D), lambda i: (0, 0)),
            pl.BlockSpec((3 * D, 2 * D), lambda i: (0, 0)),
            pl.BlockSpec((2 * D, 2 * D), lambda i: (0, 0)),
            pl.BlockSpec((1, D), lambda i: (0, 0)),
            pl.BlockSpec((2, 2 * D), lambda i: (0, 0)),
        ],
        out_specs=pl.BlockSpec((EBLK, D), lambda i: (i, 0)),
        out_shape=_sds((E, D), jnp.float32),
    )(hr, hc, rbf, wrbf, w1cat, w2diag, brbf, bcat)


def _scatter_body(ef_hbm, row_hbm, zero_hbm, out_hbm,
                  idx_v, feat_v, idx2_v, feat2_v, agg_sh, lsems, asems):
    cid = lax.axis_index("c")
    sid = lax.axis_index("s")
    rbase = sid * RPT

    @pl.when(sid < NS - 1)
    def _():
        pltpu.sync_copy(zero_hbm.at[pl.ds(rbase, RPT)],
                        agg_sh.at[pl.ds(rbase, RPT)])

    @pl.when(sid == NS - 1)
    def _():
        pltpu.sync_copy(zero_hbm.at[pl.ds(rbase, RPT_LAST)],
                        agg_sh.at[pl.ds(rbase, RPT_LAST)])

    plsc.subcore_barrier()

    base_t = cid * SC_EDGES + sid * T_EDGES

    def load(i, b):
        base = base_t + i * GCHUNK
        pltpu.sync_copy(row_hbm.at[pl.ds(base, GCHUNK)], idx_v.at[b])
        pltpu.async_copy(ef_hbm.at[pl.ds(base, GCHUNK)], feat_v.at[b],
                         lsems.at[b])

    def drain_load(b):
        pltpu.make_async_copy(ef_hbm.at[pl.ds(0, GCHUNK)], feat_v.at[b],
                              lsems.at[b]).wait()

    def fire_add(b):
        pltpu.async_copy(feat_v.at[b], agg_sh.at[idx_v.at[b]], asems.at[b],
                         add=True)

    def drain_add(b):
        pltpu.make_async_copy(feat_v.at[b], agg_sh.at[idx_v.at[b]],
                              asems.at[b]).wait()

    load(0, 0)

    def half(i, b):
        @pl.when(i + 1 < SCH_FULL)
        def _():
            @pl.when(i >= 1)
            def _():
                drain_add(1 - b)

            load(i + 1, 1 - b)

        drain_load(b)
        fire_add(b)

    def pair(k, carry):
        half(2 * k, 0)
        half(2 * k + 1, 1)
        return carry

    lax.fori_loop(0, SCH_FULL // 2, pair, 0)
    drain_add(0)
    drain_add(1)

    base = base_t + SCH_FULL * GCHUNK
    pltpu.sync_copy(row_hbm.at[pl.ds(base, SREM)], idx2_v)
    pltpu.sync_copy(ef_hbm.at[pl.ds(base, SREM)], feat2_v)
    pltpu.sync_copy(feat2_v, agg_sh.at[idx2_v], add=True)

    plsc.subcore_barrier()

    @pl.when(sid < NS - 1)
    def _():
        pltpu.sync_copy(agg_sh.at[pl.ds(rbase, RPT)],
                        out_hbm.at[cid, pl.ds(rbase, RPT)])

    @pl.when(sid == NS - 1)
    def _():
        pltpu.sync_copy(agg_sh.at[pl.ds(rbase, RPT_LAST)],
                        out_hbm.at[cid, pl.ds(rbase, RPT_LAST)])


def _scatter(edge_feat, row, zeros):
    mesh = plsc.VectorSubcoreMesh(core_axis_name="c", subcore_axis_name="s",
                                  num_cores=NC, num_subcores=NS)
    k = pl.kernel(
        _scatter_body,
        out_type=_sds((NC, N, D), jnp.float32),
        mesh=mesh,
        scratch_types=[
            pltpu.VMEM((2, GCHUNK), jnp.int32),
            pltpu.VMEM((2, GCHUNK, D), jnp.float32),
            pltpu.VMEM((SREM,), jnp.int32),
            pltpu.VMEM((SREM, D), jnp.float32),
            pltpu.VMEM_SHARED((N, D), jnp.float32),
            pltpu.SemaphoreType.DMA((2,)),
            pltpu.SemaphoreType.DMA((2,)),
        ],
        compiler_params=pltpu.CompilerParams(needs_layout_passes=False),
    )
    return k(edge_feat, row, zeros)


def _node1_body(h_ref, ap_ref, w_ref, p_ref, o_ref):
    b = p_ref[0, :]
    g = p_ref[1, :]
    be = p_ref[2, :]
    rm = p_ref[3, :]
    rv = p_ref[4, :]
    scale = g / jnp.sqrt(rv + 1e-5)
    h = h_ref[...]
    agg = ap_ref[0] + ap_ref[1]
    y = (jnp.dot(h, w_ref[0:D, :], preferred_element_type=jnp.float32)
         + jnp.dot(agg, w_ref[D:2 * D, :], preferred_element_type=jnp.float32)
         + b)
    y = _leaky(y)
    o_ref[...] = h + (y - rm) * scale + be


def _node1(h, agg_partials, w, params):
    blk = 1000
    return pl.pallas_call(
        _node1_body,
        grid=(N // blk,),
        in_specs=[
            pl.BlockSpec((blk, D), lambda i: (i, 0)),
            pl.BlockSpec((NC, blk, D), lambda i: (0, i, 0)),
            pl.BlockSpec((2 * D, D), lambda i: (0, 0)),
            pl.BlockSpec((5, D), lambda i: (0, 0)),
        ],
        out_specs=pl.BlockSpec((blk, D), lambda i: (i, 0)),
        out_shape=_sds((N, D), jnp.float32),
    )(h, agg_partials, w, params)


def kernel(x, pos, edge_index_inter, W_n0, b_n0, g0, be0, rm0, rv0,
           W_rbf, b_rbf, W_r1, b_r1, W_r2, b_r2, W_c1, b_c1, W_c2, b_c2,
           W_n, b_n, g1, be1, rm1, rv1):
    row = edge_index_inter[0]
    col = edge_index_inter[1]
    px, py, pz = pos[:, 0], pos[:, 1], pos[:, 2]
    params0 = jnp.stack([b_n0, g0, be0, rm0, rv0])
    params1 = jnp.stack([b_n, g1, be1, rm1, rv1])
    w1cat = jnp.concatenate([W_r1, W_c1], axis=1).astype(jnp.bfloat16)
    w2diag = jnp.zeros((2 * D, 2 * D), jnp.float32)
    w2diag = w2diag.at[0:D, 0:D].set(W_r2).at[D:2 * D, D:2 * D].set(W_c2)
    w2diag = w2diag.astype(jnp.bfloat16)
    brbf = b_rbf[None, :]
    bcat = jnp.stack([jnp.concatenate([b_r1, b_c1]),
                      jnp.concatenate([b_r2, b_c2])])
    zeros = jnp.zeros((N, D), jnp.float32)

    h = _node0(x, W_n0, params0)
    hr, hc, rbf = _gather(h, px, py, pz, row, col)
    edge_feat = _edge_mlp(hr, hc, rbf, W_rbf, w1cat, w2diag, brbf, bcat)
    agg_partials = _scatter(edge_feat, row, zeros)
    return _node1(h, agg_partials, W_n, params1)
